# Initial kernel scaffold; baseline (speedup 1.0000x reference)
import jax
import jax.numpy as jnp
from jax.experimental import pallas as pl


def kernel(x, edge_index, W1, b1, W2, b2):
    raise NotImplementedError("write your pallas kernel here")


# speedup vs baseline: 22.2781x; 22.2781x over previous
import functools

import jax
import jax.numpy as jnp
from jax import lax
from jax.experimental import pallas as pl
from jax.experimental.pallas import tpu as pltpu
from jax.experimental.pallas import tpu_sc as plsc

N = 10000
F = 128
E = 320000
NC = 2
NS = 16
NW = NC * NS
K = 64
EPT = 10240
CHUNKS = EPT // K
NPAD = 10240
RPT = NPAD // NS


def _sc_mesh():
    return plsc.VectorSubcoreMesh(core_axis_name="c", subcore_axis_name="s")


def _sc_degree(dst3):
    @functools.partial(
        pl.kernel,
        out_type=jax.ShapeDtypeStruct((NC, NPAD, 16), jnp.float32),
        mesh=_sc_mesh(),
        scratch_types=[
            pltpu.VMEM((CHUNKS, K), jnp.int32),
            pltpu.VMEM((K, 16), jnp.float32),
            pltpu.VMEM((K, 16), jnp.float32),
            pltpu.VMEM_SHARED((NPAD, 16), jnp.float32),
        ],
    )
    def deg_kernel(dst_hbm, out_hbm, dstv, onesb, zb, acc):
        c = lax.axis_index("c")
        s = lax.axis_index("s")
        wid = c * NS + s

        def fill(i, carry):
            onesb[i, pl.ds(0, 16)] = jnp.ones((16,), jnp.float32)
            zb[i, pl.ds(0, 16)] = jnp.zeros((16,), jnp.float32)
            return carry

        lax.fori_loop(0, K, fill, 0)
        row0 = s * RPT

        def zrow(b, carry):
            pltpu.sync_copy(zb, acc.at[pl.ds(row0 + b * K, K)])
            return carry

        lax.fori_loop(0, RPT // K, zrow, 0)
        pltpu.sync_copy(dst_hbm.at[wid], dstv)
        plsc.subcore_barrier()

        def body(j, carry):
            pltpu.sync_copy(onesb, acc.at[dstv.at[j]], add=True)
            return carry

        lax.fori_loop(0, CHUNKS, body, 0)
        plsc.subcore_barrier()
        pltpu.sync_copy(acc.at[pl.ds(row0, RPT)], out_hbm.at[c, pl.ds(row0, RPT)])

    return deg_kernel(dst3)


NB = CHUNKS // 16


def _sc_scatter(h, src4, dst4):
    @functools.partial(
        pl.kernel,
        out_type=jax.ShapeDtypeStruct((NC, NPAD, F), jnp.float32),
        mesh=_sc_mesh(),
        scratch_types=[
            pltpu.VMEM((16, K), jnp.int32),
            pltpu.VMEM((16, K), jnp.int32),
            pltpu.VMEM((K, F), jnp.float32),
            pltpu.VMEM((K, F), jnp.float32),
            pltpu.VMEM((16, F), jnp.float32),
            pltpu.VMEM_SHARED((NPAD, F), jnp.float32),
            pltpu.SemaphoreType.DMA,
            pltpu.SemaphoreType.DMA,
        ],
    )
    def scat_kernel(h_hbm, src_hbm, dst_hbm, out_hbm,
                    srcv, dstv, buf0, buf1, zb, acc, sem0, sem1):
        c = lax.axis_index("c")
        s = lax.axis_index("s")
        wid = c * NS + s

        def fill(i, carry):
            r = i // (F // 16)
            cb = (i % (F // 16)) * 16
            zb[r, pl.ds(cb, 16)] = jnp.zeros((16,), jnp.float32)
            return carry

        lax.fori_loop(0, 16 * (F // 16), fill, 0)
        row0 = s * RPT

        def zrow(b, carry):
            pltpu.sync_copy(zb, acc.at[pl.ds(row0 + b * 16, 16)])
            return carry

        lax.fori_loop(0, RPT // 16, zrow, 0)
        plsc.subcore_barrier()

        def block(bb, carry):
            pltpu.sync_copy(src_hbm.at[wid, bb], srcv)
            pltpu.sync_copy(dst_hbm.at[wid, bb], dstv)
            pltpu.async_copy(h_hbm.at[srcv.at[0]], buf0, sem0)
            pltpu.async_copy(h_hbm.at[srcv.at[1]], buf1, sem1)

            def body(g, carry2):
                j0 = 2 * g
                j1 = 2 * g + 1
                pltpu.make_async_copy(h_hbm.at[srcv.at[j0]], buf0, sem0).wait()
                pltpu.sync_copy(buf0, acc.at[dstv.at[j0]], add=True)

                @pl.when(g < 7)
                def _():
                    pltpu.async_copy(h_hbm.at[srcv.at[j0 + 2]], buf0, sem0)

                pltpu.make_async_copy(h_hbm.at[srcv.at[j1]], buf1, sem1).wait()
                pltpu.sync_copy(buf1, acc.at[dstv.at[j1]], add=True)

                @pl.when(g < 7)
                def _():
                    pltpu.async_copy(h_hbm.at[srcv.at[j1 + 2]], buf1, sem1)

                return carry2

            lax.fori_loop(0, 8, body, 0)
            return carry

        lax.fori_loop(0, NB, block, 0)
        plsc.subcore_barrier()
        pltpu.sync_copy(acc.at[pl.ds(row0, RPT)], out_hbm.at[c, pl.ds(row0, RPT)])

    return scat_kernel(h, src4, dst4)


_R = 2000


def _tc_matmul(x, W):
    def body(x_ref, w_ref, o_ref):
        o_ref[...] = jnp.dot(x_ref[...], w_ref[...],
                             preferred_element_type=jnp.float32)

    return pl.pallas_call(
        body,
        grid=(N // _R,),
        in_specs=[
            pl.BlockSpec((_R, F), lambda i: (i, 0)),
            pl.BlockSpec((F, F), lambda i: (0, 0)),
        ],
        out_specs=pl.BlockSpec((_R, F), lambda i: (i, 0)),
        out_shape=jax.ShapeDtypeStruct((N, F), jnp.float32),
    )(x, W)


def _dinv_block(dp):
    deg = dp[:, 0] + dp[:, 1] + 1.0
    return lax.rsqrt(deg)


def _tc_scale(degc, g):
    def body(d_ref, g_ref, o_ref):
        dinv = _dinv_block(d_ref[...])
        o_ref[...] = dinv[:, None] * g_ref[...]

    return pl.pallas_call(
        body,
        grid=(N // _R,),
        in_specs=[
            pl.BlockSpec((_R, NC), lambda i: (i, 0)),
            pl.BlockSpec((_R, F), lambda i: (i, 0)),
        ],
        out_specs=pl.BlockSpec((_R, F), lambda i: (i, 0)),
        out_shape=jax.ShapeDtypeStruct((N, F), jnp.float32),
    )(degc, g)


def _tc_layer2(accp, h1p, degc, W2, b1):
    def body(a_ref, h_ref, d_ref, w_ref, b_ref, o_ref):
        dinv = _dinv_block(d_ref[...])
        acc = a_ref[0] + a_ref[1] + h_ref[...]
        z = jnp.maximum(dinv[:, None] * acc + b_ref[...], 0.0)
        o_ref[...] = dinv[:, None] * jnp.dot(
            z, w_ref[...], preferred_element_type=jnp.float32)

    return pl.pallas_call(
        body,
        grid=(N // _R,),
        in_specs=[
            pl.BlockSpec((NC, _R, F), lambda i: (0, i, 0)),
            pl.BlockSpec((_R, F), lambda i: (i, 0)),
            pl.BlockSpec((_R, NC), lambda i: (i, 0)),
            pl.BlockSpec((F, F), lambda i: (0, 0)),
            pl.BlockSpec((1, F), lambda i: (0, 0)),
        ],
        out_specs=pl.BlockSpec((_R, F), lambda i: (i, 0)),
        out_shape=jax.ShapeDtypeStruct((N, F), jnp.float32),
    )(accp, h1p, degc, W2, b1)


def _tc_final(accp, h2p, degc, b2):
    def body(a_ref, h_ref, d_ref, b_ref, o_ref):
        dinv = _dinv_block(d_ref[...])
        acc = a_ref[0] + a_ref[1] + h_ref[...]
        o_ref[...] = dinv[:, None] * acc + b_ref[...]

    return pl.pallas_call(
        body,
        grid=(N // _R,),
        in_specs=[
            pl.BlockSpec((NC, _R, F), lambda i: (0, i, 0)),
            pl.BlockSpec((_R, F), lambda i: (i, 0)),
            pl.BlockSpec((_R, NC), lambda i: (i, 0)),
            pl.BlockSpec((1, F), lambda i: (0, 0)),
        ],
        out_specs=pl.BlockSpec((_R, F), lambda i: (i, 0)),
        out_shape=jax.ShapeDtypeStruct((N, F), jnp.float32),
    )(accp, h2p, degc, b2)


def kernel(x, edge_index, W1, b1, W2, b2):
    ei = edge_index.astype(jnp.int32)
    pad = NW * EPT - E
    ar = jnp.arange(pad, dtype=jnp.int32)
    src3 = jnp.concatenate([ei[0], ar % N]).reshape(NW, CHUNKS, K)
    dst3 = jnp.concatenate([ei[1], N + ar % (NPAD - N)]).reshape(NW, CHUNKS, K)
    src4 = src3.reshape(NW, NB, 16, K)
    dst4 = dst3.reshape(NW, NB, 16, K)

    degp = _sc_degree(dst3)
    degc = degp[:, :, 0].T

    g1 = _tc_matmul(x, W1)
    h1p = _tc_scale(degc, g1)
    acc1 = _sc_scatter(h1p, src4, dst4)
    h2p = _tc_layer2(acc1, h1p, degc, W2, b1.reshape(1, F))
    acc2 = _sc_scatter(h2p, src4, dst4)
    return _tc_final(acc2, h2p, degc, b2.reshape(1, F))


# speedup vs baseline: 25.1714x; 1.1299x over previous
import functools

import jax
import jax.numpy as jnp
from jax import lax
from jax.experimental import pallas as pl
from jax.experimental.pallas import tpu as pltpu
from jax.experimental.pallas import tpu_sc as plsc

N = 10000
F = 128
E = 320000
NC = 2
NS = 16
NW = NC * NS
K = 64
EPT = 10240
CHUNKS = EPT // K
NPAD = 10240
RPT = NPAD // NS


def _sc_mesh():
    return plsc.VectorSubcoreMesh(core_axis_name="c", subcore_axis_name="s")


_DK = 128
_DCH = EPT // _DK


def _sc_degree(dst2):
    @functools.partial(
        pl.kernel,
        out_type=jax.ShapeDtypeStruct((NC, NPAD, 16), jnp.float32),
        mesh=_sc_mesh(),
        scratch_types=[
            pltpu.VMEM((_DCH, _DK), jnp.int32),
            pltpu.VMEM((_DK, 16), jnp.float32),
            pltpu.VMEM((_DK, 16), jnp.float32),
            pltpu.VMEM_SHARED((NPAD, 16), jnp.float32),
            pltpu.SemaphoreType.DMA,
        ],
    )
    def deg_kernel(dst_hbm, out_hbm, dstv, onesb, zb, acc, sem):
        c = lax.axis_index("c")
        s = lax.axis_index("s")
        wid = c * NS + s

        def fill(i, carry):
            onesb[i, pl.ds(0, 16)] = jnp.ones((16,), jnp.float32)
            zb[i, pl.ds(0, 16)] = jnp.zeros((16,), jnp.float32)
            return carry

        lax.fori_loop(0, _DK, fill, 0)
        row0 = s * RPT

        def zrow(b, carry):
            pltpu.sync_copy(zb, acc.at[pl.ds(row0 + b * _DK, _DK)])
            return carry

        lax.fori_loop(0, RPT // _DK, zrow, 0)
        pltpu.sync_copy(dst_hbm.at[wid], dstv)
        plsc.subcore_barrier()

        def group(g, carry):
            def fire(j, carry2):
                pltpu.async_copy(onesb, acc.at[dstv.at[g * 8 + j]], sem, add=True)
                return carry2

            lax.fori_loop(0, 8, fire, 0)

            def drain(j, carry2):
                pltpu.make_async_copy(onesb, acc.at[dstv.at[g * 8 + j]], sem).wait()
                return carry2

            lax.fori_loop(0, 8, drain, 0)
            return carry

        lax.fori_loop(0, _DCH // 8, group, 0)
        plsc.subcore_barrier()
        pltpu.sync_copy(acc.at[pl.ds(row0, RPT)], out_hbm.at[c, pl.ds(row0, RPT)])

    return deg_kernel(dst2)


NB = CHUNKS // 16


def _sc_scatter(h, src4, dst4):
    @functools.partial(
        pl.kernel,
        out_type=jax.ShapeDtypeStruct((NC, NPAD, F), jnp.float32),
        mesh=_sc_mesh(),
        scratch_types=[
            pltpu.VMEM((16, K), jnp.int32),
            pltpu.VMEM((16, K), jnp.int32),
            [pltpu.VMEM((K, F), jnp.float32)] * 4,
            pltpu.VMEM((32, F), jnp.float32),
            pltpu.VMEM_SHARED((NPAD, F), jnp.float32),
            [pltpu.SemaphoreType.DMA] * 4,
            [pltpu.SemaphoreType.DMA] * 4,
        ],
    )
    def scat_kernel(h_hbm, src_hbm, dst_hbm, out_hbm,
                    srcv, dstv, bufs, zb, acc, gsem, ssem):
        c = lax.axis_index("c")
        s = lax.axis_index("s")
        wid = c * NS + s

        def fill(i, carry):
            r = i // (F // 16)
            cb = (i % (F // 16)) * 16
            zb[r, pl.ds(cb, 16)] = jnp.zeros((16,), jnp.float32)
            return carry

        lax.fori_loop(0, 32 * (F // 16), fill, 0)
        row0 = s * RPT

        def zrow(b, carry):
            pltpu.sync_copy(zb, acc.at[pl.ds(row0 + b * 32, 32)])
            return carry

        lax.fori_loop(0, RPT // 32, zrow, 0)
        plsc.subcore_barrier()

        def block(bb, carry):
            pltpu.sync_copy(src_hbm.at[wid, bb], srcv)
            pltpu.sync_copy(dst_hbm.at[wid, bb], dstv)
            pltpu.async_copy(h_hbm.at[srcv.at[0]], bufs[0], gsem[0])
            pltpu.async_copy(h_hbm.at[srcv.at[1]], bufs[1], gsem[1])

            def body(g, carry2):
                for b in range(4):
                    j = 4 * g + b
                    b2 = (b + 2) % 4

                    @pl.when(j >= 2)
                    def _():
                        pltpu.make_async_copy(
                            bufs[b2], acc.at[dstv.at[j - 2]], ssem[b2]).wait()

                    @pl.when(j + 2 < 16)
                    def _():
                        pltpu.async_copy(
                            h_hbm.at[srcv.at[j + 2]], bufs[b2], gsem[b2])

                    pltpu.make_async_copy(
                        h_hbm.at[srcv.at[j]], bufs[b], gsem[b]).wait()
                    pltpu.async_copy(
                        bufs[b], acc.at[dstv.at[j]], ssem[b], add=True)
                return carry2

            lax.fori_loop(0, 4, body, 0)
            pltpu.make_async_copy(bufs[2], acc.at[dstv.at[14]], ssem[2]).wait()
            pltpu.make_async_copy(bufs[3], acc.at[dstv.at[15]], ssem[3]).wait()
            return carry

        lax.fori_loop(0, NB, block, 0)
        plsc.subcore_barrier()
        pltpu.sync_copy(acc.at[pl.ds(row0, RPT)], out_hbm.at[c, pl.ds(row0, RPT)])

    return scat_kernel(h, src4, dst4)


_R = 2000


def _tc_matmul(x, W):
    def body(x_ref, w_ref, o_ref):
        o_ref[...] = jnp.dot(x_ref[...], w_ref[...],
                             preferred_element_type=jnp.float32)

    return pl.pallas_call(
        body,
        grid=(N // _R,),
        in_specs=[
            pl.BlockSpec((_R, F), lambda i: (i, 0)),
            pl.BlockSpec((F, F), lambda i: (0, 0)),
        ],
        out_specs=pl.BlockSpec((_R, F), lambda i: (i, 0)),
        out_shape=jax.ShapeDtypeStruct((N, F), jnp.float32),
    )(x, W)


def _dinv_block(dp):
    deg = dp[:, 0] + dp[:, 1] + 1.0
    return lax.rsqrt(deg)


def _tc_scale(degc, g):
    def body(d_ref, g_ref, o_ref):
        dinv = _dinv_block(d_ref[...])
        o_ref[...] = dinv[:, None] * g_ref[...]

    return pl.pallas_call(
        body,
        grid=(N // _R,),
        in_specs=[
            pl.BlockSpec((_R, NC), lambda i: (i, 0)),
            pl.BlockSpec((_R, F), lambda i: (i, 0)),
        ],
        out_specs=pl.BlockSpec((_R, F), lambda i: (i, 0)),
        out_shape=jax.ShapeDtypeStruct((N, F), jnp.float32),
    )(degc, g)


def _tc_layer2(accp, h1p, degc, W2, b1):
    def body(a_ref, h_ref, d_ref, w_ref, b_ref, o_ref):
        dinv = _dinv_block(d_ref[...])
        acc = a_ref[0] + a_ref[1] + h_ref[...]
        z = jnp.maximum(dinv[:, None] * acc + b_ref[...], 0.0)
        o_ref[...] = dinv[:, None] * jnp.dot(
            z, w_ref[...], preferred_element_type=jnp.float32)

    return pl.pallas_call(
        body,
        grid=(N // _R,),
        in_specs=[
            pl.BlockSpec((NC, _R, F), lambda i: (0, i, 0)),
            pl.BlockSpec((_R, F), lambda i: (i, 0)),
            pl.BlockSpec((_R, NC), lambda i: (i, 0)),
            pl.BlockSpec((F, F), lambda i: (0, 0)),
            pl.BlockSpec((1, F), lambda i: (0, 0)),
        ],
        out_specs=pl.BlockSpec((_R, F), lambda i: (i, 0)),
        out_shape=jax.ShapeDtypeStruct((N, F), jnp.float32),
    )(accp, h1p, degc, W2, b1)


def _tc_final(accp, h2p, degc, b2):
    def body(a_ref, h_ref, d_ref, b_ref, o_ref):
        dinv = _dinv_block(d_ref[...])
        acc = a_ref[0] + a_ref[1] + h_ref[...]
        o_ref[...] = dinv[:, None] * acc + b_ref[...]

    return pl.pallas_call(
        body,
        grid=(N // _R,),
        in_specs=[
            pl.BlockSpec((NC, _R, F), lambda i: (0, i, 0)),
            pl.BlockSpec((_R, F), lambda i: (i, 0)),
            pl.BlockSpec((_R, NC), lambda i: (i, 0)),
            pl.BlockSpec((1, F), lambda i: (0, 0)),
        ],
        out_specs=pl.BlockSpec((_R, F), lambda i: (i, 0)),
        out_shape=jax.ShapeDtypeStruct((N, F), jnp.float32),
    )(accp, h2p, degc, b2)


def kernel(x, edge_index, W1, b1, W2, b2):
    ei = edge_index.astype(jnp.int32)
    pad = NW * EPT - E
    ar = jnp.arange(pad, dtype=jnp.int32)
    src_flat = jnp.concatenate([ei[0], ar % N])
    dst_flat = jnp.concatenate([ei[1], N + ar % (NPAD - N)])
    src4 = src_flat.reshape(NW, NB, 16, K)
    dst4 = dst_flat.reshape(NW, NB, 16, K)

    degp = _sc_degree(dst_flat.reshape(NW, _DCH, _DK))
    degc = degp[:, :, 0].T

    g1 = _tc_matmul(x, W1)
    h1p = _tc_scale(degc, g1)
    acc1 = _sc_scatter(h1p, src4, dst4)
    h2p = _tc_layer2(acc1, h1p, degc, W2, b1.reshape(1, F))
    acc2 = _sc_scatter(h2p, src4, dst4)
    return _tc_final(acc2, h2p, degc, b2.reshape(1, F))


# speedup vs baseline: 26.5805x; 1.0560x over previous
import functools

import jax
import jax.numpy as jnp
from jax import lax
from jax.experimental import pallas as pl
from jax.experimental.pallas import tpu as pltpu
from jax.experimental.pallas import tpu_sc as plsc

N = 10000
F = 128
E = 320000
NC = 2
NS = 16
NW = NC * NS
K = 64
EPT = 10240
CHUNKS = EPT // K
NPAD = 10240
RPT = NPAD // NS


def _sc_mesh():
    return plsc.VectorSubcoreMesh(core_axis_name="c", subcore_axis_name="s")


_DK = 128
_DCH = EPT // _DK


def _sc_degree(dst2):
    @functools.partial(
        pl.kernel,
        out_type=jax.ShapeDtypeStruct((NC, NPAD, 16), jnp.float32),
        mesh=_sc_mesh(),
        scratch_types=[
            pltpu.VMEM((_DCH, _DK), jnp.int32),
            pltpu.VMEM((_DK, 16), jnp.float32),
            pltpu.VMEM((_DK, 16), jnp.float32),
            pltpu.VMEM_SHARED((NPAD, 16), jnp.float32),
            pltpu.SemaphoreType.DMA,
        ],
    )
    def deg_kernel(dst_hbm, out_hbm, dstv, onesb, zb, acc, sem):
        c = lax.axis_index("c")
        s = lax.axis_index("s")
        wid = c * NS + s

        def fill(i, carry):
            onesb[i, pl.ds(0, 16)] = jnp.ones((16,), jnp.float32)
            zb[i, pl.ds(0, 16)] = jnp.zeros((16,), jnp.float32)
            return carry

        lax.fori_loop(0, _DK, fill, 0)
        row0 = s * RPT

        def zrow(b, carry):
            pltpu.sync_copy(zb, acc.at[pl.ds(row0 + b * _DK, _DK)])
            return carry

        lax.fori_loop(0, RPT // _DK, zrow, 0)
        pltpu.sync_copy(dst_hbm.at[wid], dstv)
        plsc.subcore_barrier()

        def group(g, carry):
            def fire(j, carry2):
                pltpu.async_copy(onesb, acc.at[dstv.at[g * 8 + j]], sem, add=True)
                return carry2

            lax.fori_loop(0, 8, fire, 0)

            def drain(j, carry2):
                pltpu.make_async_copy(onesb, acc.at[dstv.at[g * 8 + j]], sem).wait()
                return carry2

            lax.fori_loop(0, 8, drain, 0)
            return carry

        lax.fori_loop(0, _DCH // 8, group, 0)
        plsc.subcore_barrier()
        pltpu.sync_copy(acc.at[pl.ds(row0, RPT)], out_hbm.at[c, pl.ds(row0, RPT)])

    return deg_kernel(dst2)


NB = CHUNKS // 16


def _sc_scatter(h, pairs):
    @functools.partial(
        pl.kernel,
        out_type=jax.ShapeDtypeStruct((NC, NPAD, F), jnp.float32),
        mesh=_sc_mesh(),
        scratch_types=[
            [pltpu.VMEM((32, K), jnp.int32)] * 2,
            [pltpu.VMEM((K, F), jnp.float32)] * 4,
            pltpu.VMEM((32, F), jnp.float32),
            pltpu.VMEM_SHARED((NPAD, F), jnp.float32),
            [pltpu.SemaphoreType.DMA] * 4,
            [pltpu.SemaphoreType.DMA] * 4,
            pltpu.SemaphoreType.DMA,
        ],
    )
    def scat_kernel(h_hbm, pairs_hbm, out_hbm,
                    idx, bufs, zb, acc, gsem, ssem, isem):
        c = lax.axis_index("c")
        s = lax.axis_index("s")
        wid = c * NS + s

        def fill(i, carry):
            r = i // (F // 16)
            cb = (i % (F // 16)) * 16
            zb[r, pl.ds(cb, 16)] = jnp.zeros((16,), jnp.float32)
            return carry

        lax.fori_loop(0, 32 * (F // 16), fill, 0)
        row0 = s * RPT

        def zrow(b, carry):
            pltpu.sync_copy(zb, acc.at[pl.ds(row0 + b * 32, 32)])
            return carry

        lax.fori_loop(0, RPT // 32, zrow, 0)
        plsc.subcore_barrier()

        pltpu.sync_copy(pairs_hbm.at[wid, 0], idx[0])
        pltpu.async_copy(h_hbm.at[idx[0].at[0]], bufs[0], gsem[0])
        pltpu.async_copy(h_hbm.at[idx[0].at[1]], bufs[1], gsem[1])

        def superblock(q, carry):
            for half in range(2):
                bb = 2 * q + half
                cur = idx[half]
                nxt = idx[1 - half]

                @pl.when(bb + 1 < NB)
                def _():
                    pltpu.async_copy(pairs_hbm.at[wid, bb + 1], nxt, isem)

                def body(g, carry2):
                    for b in range(4):
                        j = 4 * g + b
                        b2 = (b + 2) % 4

                        @pl.when(j >= 2)
                        def _():
                            pltpu.make_async_copy(
                                bufs[b2], acc.at[cur.at[16 + j - 2]],
                                ssem[b2]).wait()

                        @pl.when(j + 2 < 16)
                        def _():
                            pltpu.async_copy(
                                h_hbm.at[cur.at[j + 2]], bufs[b2], gsem[b2])

                        pltpu.make_async_copy(
                            h_hbm.at[cur.at[j]], bufs[b], gsem[b]).wait()
                        pltpu.async_copy(
                            bufs[b], acc.at[cur.at[16 + j]], ssem[b], add=True)
                    return carry2

                lax.fori_loop(0, 4, body, 0)
                pltpu.make_async_copy(
                    bufs[2], acc.at[cur.at[30]], ssem[2]).wait()
                pltpu.make_async_copy(
                    bufs[3], acc.at[cur.at[31]], ssem[3]).wait()

                @pl.when(bb + 1 < NB)
                def _():
                    pltpu.make_async_copy(
                        pairs_hbm.at[wid, bb + 1], nxt, isem).wait()
                    pltpu.async_copy(h_hbm.at[nxt.at[0]], bufs[0], gsem[0])
                    pltpu.async_copy(h_hbm.at[nxt.at[1]], bufs[1], gsem[1])

            return carry

        lax.fori_loop(0, NB // 2, superblock, 0)
        plsc.subcore_barrier()
        pltpu.sync_copy(acc.at[pl.ds(row0, RPT)], out_hbm.at[c, pl.ds(row0, RPT)])

    return scat_kernel(h, pairs)


_R = 2000


def _dinv_block(dp):
    deg = dp[:, 0] + dp[:, 1] + 1.0
    return lax.rsqrt(deg)


def _tc_layer1(degc, x, W):
    def body(d_ref, x_ref, w_ref, o_ref):
        dinv = _dinv_block(d_ref[...])
        o_ref[...] = dinv[:, None] * jnp.dot(
            x_ref[...], w_ref[...], preferred_element_type=jnp.float32)

    return pl.pallas_call(
        body,
        grid=(N // _R,),
        in_specs=[
            pl.BlockSpec((_R, NC), lambda i: (i, 0)),
            pl.BlockSpec((_R, F), lambda i: (i, 0)),
            pl.BlockSpec((F, F), lambda i: (0, 0)),
        ],
        out_specs=pl.BlockSpec((_R, F), lambda i: (i, 0)),
        out_shape=jax.ShapeDtypeStruct((N, F), jnp.float32),
    )(degc, x, W)


def _tc_layer2(accp, h1p, degc, W2, b1):
    def body(a_ref, h_ref, d_ref, w_ref, b_ref, o_ref):
        dinv = _dinv_block(d_ref[...])
        acc = a_ref[0] + a_ref[1] + h_ref[...]
        z = jnp.maximum(dinv[:, None] * acc + b_ref[...], 0.0)
        o_ref[...] = dinv[:, None] * jnp.dot(
            z, w_ref[...], preferred_element_type=jnp.float32)

    return pl.pallas_call(
        body,
        grid=(N // _R,),
        in_specs=[
            pl.BlockSpec((NC, _R, F), lambda i: (0, i, 0)),
            pl.BlockSpec((_R, F), lambda i: (i, 0)),
            pl.BlockSpec((_R, NC), lambda i: (i, 0)),
            pl.BlockSpec((F, F), lambda i: (0, 0)),
            pl.BlockSpec((1, F), lambda i: (0, 0)),
        ],
        out_specs=pl.BlockSpec((_R, F), lambda i: (i, 0)),
        out_shape=jax.ShapeDtypeStruct((N, F), jnp.float32),
    )(accp, h1p, degc, W2, b1)


def _tc_final(accp, h2p, degc, b2):
    def body(a_ref, h_ref, d_ref, b_ref, o_ref):
        dinv = _dinv_block(d_ref[...])
        acc = a_ref[0] + a_ref[1] + h_ref[...]
        o_ref[...] = dinv[:, None] * acc + b_ref[...]

    return pl.pallas_call(
        body,
        grid=(N // _R,),
        in_specs=[
            pl.BlockSpec((NC, _R, F), lambda i: (0, i, 0)),
            pl.BlockSpec((_R, F), lambda i: (i, 0)),
            pl.BlockSpec((_R, NC), lambda i: (i, 0)),
            pl.BlockSpec((1, F), lambda i: (0, 0)),
        ],
        out_specs=pl.BlockSpec((_R, F), lambda i: (i, 0)),
        out_shape=jax.ShapeDtypeStruct((N, F), jnp.float32),
    )(accp, h2p, degc, b2)


def kernel(x, edge_index, W1, b1, W2, b2):
    ei = edge_index.astype(jnp.int32)
    pad = NW * EPT - E
    ar = jnp.arange(pad, dtype=jnp.int32)
    src_flat = jnp.concatenate([ei[0], ar % N])
    dst_flat = jnp.concatenate([ei[1], N + ar % (NPAD - N)])
    src4 = src_flat.reshape(NW, NB, 16, K)
    dst4 = dst_flat.reshape(NW, NB, 16, K)
    pairs = jnp.concatenate([src4, dst4], axis=2)

    degp = _sc_degree(dst_flat.reshape(NW, _DCH, _DK))
    degc = degp[:, :, 0].T

    h1p = _tc_layer1(degc, x, W1)
    acc1 = _sc_scatter(h1p, pairs)
    h2p = _tc_layer2(acc1, h1p, degc, W2, b1.reshape(1, F))
    acc2 = _sc_scatter(h2p, pairs)
    return _tc_final(acc2, h2p, degc, b2.reshape(1, F))


# speedup vs baseline: 31.8490x; 1.1982x over previous
import functools

import jax
import jax.numpy as jnp
from jax import lax
from jax.experimental import pallas as pl
from jax.experimental.pallas import tpu as pltpu
from jax.experimental.pallas import tpu_sc as plsc

N = 10000
F = 128
E = 320000
NC = 2
NS = 16
NW = NC * NS
K = 64
EPT = 10240
CHUNKS = EPT // K
NPAD = 10240
RPT = NPAD // NS


def _sc_mesh():
    return plsc.VectorSubcoreMesh(core_axis_name="c", subcore_axis_name="s")


_DK = 128
_DCH = EPT // _DK


def _sc_degree(dst2):
    @functools.partial(
        pl.kernel,
        out_type=jax.ShapeDtypeStruct((NC, NPAD, 16), jnp.float32),
        mesh=_sc_mesh(),
        scratch_types=[
            pltpu.VMEM((_DCH, _DK), jnp.int32),
            pltpu.VMEM((_DK, 16), jnp.float32),
            pltpu.VMEM((_DK, 16), jnp.float32),
            pltpu.VMEM_SHARED((NPAD, 16), jnp.float32),
            pltpu.SemaphoreType.DMA,
        ],
    )
    def deg_kernel(dst_hbm, out_hbm, dstv, onesb, zb, acc, sem):
        c = lax.axis_index("c")
        s = lax.axis_index("s")
        wid = c * NS + s

        def fill(i, carry):
            onesb[i, pl.ds(0, 16)] = jnp.ones((16,), jnp.float32)
            zb[i, pl.ds(0, 16)] = jnp.zeros((16,), jnp.float32)
            return carry

        lax.fori_loop(0, _DK, fill, 0)
        row0 = s * RPT

        def zrow(b, carry):
            pltpu.sync_copy(zb, acc.at[pl.ds(row0 + b * _DK, _DK)])
            return carry

        lax.fori_loop(0, RPT // _DK, zrow, 0)
        pltpu.sync_copy(dst_hbm.at[wid], dstv)
        plsc.subcore_barrier()

        def group(g, carry):
            def fire(j, carry2):
                pltpu.async_copy(onesb, acc.at[dstv.at[g * 8 + j]], sem, add=True)
                return carry2

            lax.fori_loop(0, 8, fire, 0)

            def drain(j, carry2):
                pltpu.make_async_copy(onesb, acc.at[dstv.at[g * 8 + j]], sem).wait()
                return carry2

            lax.fori_loop(0, 8, drain, 0)
            return carry

        lax.fori_loop(0, _DCH // 8, group, 0)
        plsc.subcore_barrier()
        pltpu.sync_copy(acc.at[pl.ds(row0, RPT)], out_hbm.at[c, pl.ds(row0, RPT)])

    return deg_kernel(dst2)


NB = CHUNKS // 16


def _sc_scatter(h, pairs):
    @functools.partial(
        pl.kernel,
        out_type=jax.ShapeDtypeStruct((NC, NPAD, F), jnp.float32),
        mesh=_sc_mesh(),
        scratch_types=[
            [pltpu.VMEM((32, K), jnp.int32)] * 2,
            [pltpu.VMEM((K, F), jnp.float32)] * 4,
            pltpu.VMEM((32, F), jnp.float32),
            pltpu.VMEM_SHARED((NPAD, F), jnp.float32),
            [pltpu.SemaphoreType.DMA] * 4,
            [pltpu.SemaphoreType.DMA] * 4,
            pltpu.SemaphoreType.DMA,
        ],
    )
    def scat_kernel(h_hbm, pairs_hbm, out_hbm,
                    idx, bufs, zb, acc, gsem, ssem, isem):
        c = lax.axis_index("c")
        s = lax.axis_index("s")
        wid = c * NS + s

        def fill(i, carry):
            r = i // (F // 16)
            cb = (i % (F // 16)) * 16
            zb[r, pl.ds(cb, 16)] = jnp.zeros((16,), jnp.float32)
            return carry

        lax.fori_loop(0, 32 * (F // 16), fill, 0)
        row0 = s * RPT

        def zrow(b, carry):
            pltpu.sync_copy(zb, acc.at[pl.ds(row0 + b * 32, 32)])
            return carry

        lax.fori_loop(0, RPT // 32, zrow, 0)
        plsc.subcore_barrier()

        pltpu.sync_copy(pairs_hbm.at[wid, 0], idx[0])
        pltpu.async_copy(h_hbm.at[idx[0].at[0]], bufs[0], gsem[0])
        pltpu.async_copy(h_hbm.at[idx[0].at[1]], bufs[1], gsem[1])

        def superblock(q, carry):
            for half in range(2):
                bb = 2 * q + half
                cur = idx[half]
                nxt = idx[1 - half]

                @pl.when(bb + 1 < NB)
                def _():
                    pltpu.async_copy(pairs_hbm.at[wid, bb + 1], nxt, isem)

                def body(g, carry2):
                    for b in range(4):
                        j = 4 * g + b
                        b2 = (b + 2) % 4

                        @pl.when(j >= 2)
                        def _():
                            pltpu.make_async_copy(
                                bufs[b2], acc.at[cur.at[16 + j - 2]],
                                ssem[b2]).wait()

                        @pl.when(j + 2 < 16)
                        def _():
                            pltpu.async_copy(
                                h_hbm.at[cur.at[j + 2]], bufs[b2], gsem[b2])

                        pltpu.make_async_copy(
                            h_hbm.at[cur.at[j]], bufs[b], gsem[b]).wait()
                        pltpu.async_copy(
                            bufs[b], acc.at[cur.at[16 + j]], ssem[b], add=True)
                    return carry2

                lax.fori_loop(0, 4, body, 0)
                pltpu.make_async_copy(
                    bufs[2], acc.at[cur.at[30]], ssem[2]).wait()
                pltpu.make_async_copy(
                    bufs[3], acc.at[cur.at[31]], ssem[3]).wait()

                @pl.when(bb + 1 < NB)
                def _():
                    pltpu.make_async_copy(
                        pairs_hbm.at[wid, bb + 1], nxt, isem).wait()
                    pltpu.async_copy(h_hbm.at[nxt.at[0]], bufs[0], gsem[0])
                    pltpu.async_copy(h_hbm.at[nxt.at[1]], bufs[1], gsem[1])

            return carry

        lax.fori_loop(0, NB // 2, superblock, 0)
        plsc.subcore_barrier()
        pltpu.sync_copy(acc.at[pl.ds(row0, RPT)], out_hbm.at[c, pl.ds(row0, RPT)])

    return scat_kernel(h, pairs)


_R = 2000


def _dinv_block(dp):
    deg = dp[0, :, 0] + dp[1, :, 0] + 1.0
    return lax.rsqrt(deg)


_DEG_SPEC = pl.BlockSpec((NC, _R, 16), lambda i: (0, i, 0))


def _tc_layer1(degc, x, W):
    def body(d_ref, x_ref, w_ref, o_ref):
        dinv = _dinv_block(d_ref[...])
        o_ref[...] = dinv[:, None] * jnp.dot(
            x_ref[...], w_ref[...], preferred_element_type=jnp.float32)

    return pl.pallas_call(
        body,
        grid=(N // _R,),
        in_specs=[
            _DEG_SPEC,
            pl.BlockSpec((_R, F), lambda i: (i, 0)),
            pl.BlockSpec((F, F), lambda i: (0, 0)),
        ],
        out_specs=pl.BlockSpec((_R, F), lambda i: (i, 0)),
        out_shape=jax.ShapeDtypeStruct((N, F), jnp.float32),
    )(degc, x, W)


def _tc_layer2(accp, h1p, degc, W2, b1):
    def body(a_ref, h_ref, d_ref, w_ref, b_ref, o_ref):
        dinv = _dinv_block(d_ref[...])
        acc = a_ref[0] + a_ref[1] + h_ref[...]
        z = jnp.maximum(dinv[:, None] * acc + b_ref[...], 0.0)
        o_ref[...] = dinv[:, None] * jnp.dot(
            z, w_ref[...], preferred_element_type=jnp.float32)

    return pl.pallas_call(
        body,
        grid=(N // _R,),
        in_specs=[
            pl.BlockSpec((NC, _R, F), lambda i: (0, i, 0)),
            pl.BlockSpec((_R, F), lambda i: (i, 0)),
            _DEG_SPEC,
            pl.BlockSpec((F, F), lambda i: (0, 0)),
            pl.BlockSpec((1, F), lambda i: (0, 0)),
        ],
        out_specs=pl.BlockSpec((_R, F), lambda i: (i, 0)),
        out_shape=jax.ShapeDtypeStruct((N, F), jnp.float32),
    )(accp, h1p, degc, W2, b1)


def _tc_final(accp, h2p, degc, b2):
    def body(a_ref, h_ref, d_ref, b_ref, o_ref):
        dinv = _dinv_block(d_ref[...])
        acc = a_ref[0] + a_ref[1] + h_ref[...]
        o_ref[...] = dinv[:, None] * acc + b_ref[...]

    return pl.pallas_call(
        body,
        grid=(N // _R,),
        in_specs=[
            pl.BlockSpec((NC, _R, F), lambda i: (0, i, 0)),
            pl.BlockSpec((_R, F), lambda i: (i, 0)),
            _DEG_SPEC,
            pl.BlockSpec((1, F), lambda i: (0, 0)),
        ],
        out_specs=pl.BlockSpec((_R, F), lambda i: (i, 0)),
        out_shape=jax.ShapeDtypeStruct((N, F), jnp.float32),
    )(accp, h2p, degc, b2)


def kernel(x, edge_index, W1, b1, W2, b2):
    ei = edge_index.astype(jnp.int32)
    pad = NW * EPT - E
    ar = jnp.arange(pad, dtype=jnp.int32)
    src_flat = jnp.concatenate([ei[0], ar % N])
    dst_flat = jnp.concatenate([ei[1], N + ar % (NPAD - N)])
    src4 = src_flat.reshape(NW, NB, 16, K)
    dst4 = dst_flat.reshape(NW, NB, 16, K)
    pairs = jnp.concatenate([src4, dst4], axis=2)

    degp = _sc_degree(dst_flat.reshape(NW, _DCH, _DK))

    h1p = _tc_layer1(degp, x, W1)
    acc1 = _sc_scatter(h1p, pairs)
    h2p = _tc_layer2(acc1, h1p, degp, W2, b1.reshape(1, F))
    acc2 = _sc_scatter(h2p, pairs)
    return _tc_final(acc2, h2p, degp, b2.reshape(1, F))
